# Initial kernel scaffold; baseline (speedup 1.0000x reference)
import jax
import jax.numpy as jnp
from jax.experimental import pallas as pl


def kernel(h, edge_index):
    raise NotImplementedError("write your pallas kernel here")


# speedup vs baseline: 2.1829x; 2.1829x over previous
import functools

import jax
import jax.numpy as jnp
from jax import lax
from jax.experimental import pallas as pl
from jax.experimental.pallas import tpu as pltpu
from jax.experimental.pallas import tpu_sc as plsc

_NC = 2
_NS = 16
_NW = _NC * _NS
_L = 16

_C = 128
_D = 128
_DB = _D // _L


def _sc_dot(n_chunks: int):
  per_w = n_chunks * _C

  mesh = plsc.VectorSubcoreMesh(core_axis_name="c", subcore_axis_name="s")

  @functools.partial(
      pl.kernel,
      out_type=jax.ShapeDtypeStruct((_NW * per_w,), jnp.float32),
      mesh=mesh,
      compiler_params=pltpu.CompilerParams(needs_layout_passes=False),
      scratch_types=[
          pltpu.VMEM((_C,), jnp.int32),
          pltpu.VMEM((_C,), jnp.int32),
          pltpu.VMEM((_C, _D), jnp.float32),
          pltpu.VMEM((_C, _D), jnp.float32),
          pltpu.VMEM((_L * _L,), jnp.float32),
          pltpu.VMEM((_C,), jnp.float32),
          pltpu.SemaphoreType.DMA,
          pltpu.SemaphoreType.DMA,
      ],
  )
  def k(h_hbm, src_hbm, dst_hbm, out_hbm,
        sidx_v, didx_v, srows_v, drows_v, pbuf_v, out_v, sem_s, sem_d):
    wid = lax.axis_index("s") * _NC + lax.axis_index("c")
    lane = lax.iota(jnp.int32, _L)

    def chunk_body(c, _):
      base = wid * per_w + c * _C
      pltpu.sync_copy(src_hbm.at[pl.ds(base, _C)], sidx_v)
      pltpu.sync_copy(dst_hbm.at[pl.ds(base, _C)], didx_v)
      cp_s = pltpu.make_async_copy(h_hbm.at[sidx_v], srows_v, sem_s)
      cp_d = pltpu.make_async_copy(h_hbm.at[didx_v], drows_v, sem_d)
      cp_s.start()
      cp_d.start()
      cp_s.wait()
      cp_d.wait()

      def group_body(g, _):
        e0 = g * _L
        for i in range(_L):
          e = e0 + i
          acc = srows_v[e, pl.ds(0, _L)] * drows_v[e, pl.ds(0, _L)]
          for j in range(1, _DB):
            acc = acc + (srows_v[e, pl.ds(j * _L, _L)] *
                         drows_v[e, pl.ds(j * _L, _L)])
          pbuf_v[pl.ds(i * _L, _L)] = acc
        lane16 = lane * _L
        sc = plsc.load_gather(pbuf_v, [lane16])
        for j in range(1, _L):
          sc = sc + plsc.load_gather(pbuf_v, [lane16 + j])
        out_v[pl.ds(e0, _L)] = sc
        return 0

      lax.fori_loop(0, _C // _L, group_body, 0)
      pltpu.sync_copy(out_v, out_hbm.at[pl.ds(base, _C)])
      return 0

    lax.fori_loop(0, n_chunks, chunk_body, 0)

  return k


def kernel(h, edge_index):
  B = edge_index.shape[1]
  n_chunks = -(-B // (_NW * _C))
  Bp = _NW * _C * n_chunks
  src = edge_index[0]
  dst = edge_index[1]
  if Bp != B:
    src = jnp.pad(src, (0, Bp - B))
    dst = jnp.pad(dst, (0, Bp - B))
  out = _sc_dot(n_chunks)(h, src, dst)
  return out[:B]
